# baseline (speedup 1.0000x reference)
import functools

import jax
import jax.numpy as jnp
from jax import lax
from jax.experimental import pallas as pl
from jax.experimental.pallas import tpu as pltpu
from jax.experimental.pallas import tpu_sc as plsc

N = 10000
NP = 10240
B = 16
H = 8
D = 128
ROW = 144
E_TOT = 330000
NWORK = 32
CHUNK = 10368
ETP = NWORK * CHUNK
KBLK = 64
NBLK = CHUNK // KBLK
NPAIR_A = NBLK // 2
CHUNK16 = ETP // 16
KBLK_G = 48
NPAIR_G = CHUNK16 // KBLK_G // 2
ROW2 = 80
NTAB = 10048
RPS = NP // 16
ALPHA = 0.1
KAPPNP = 5

_MESH = plsc.VectorSubcoreMesh(core_axis_name="c", subcore_axis_name="s")


def _tk_encoder(xJp, Wp, b, mask):
    def body(x_ref, w_ref, b_ref, m_ref, o_ref):
        o_ref[...] = (x_ref[...] @ w_ref[...] + b_ref[...]) * m_ref[...]
    return pl.pallas_call(
        body, out_shape=jax.ShapeDtypeStruct((NP, D), jnp.float32),
    )(xJp, Wp, b[None], mask)


def _tk_gat_pre(h, gatW, AB):
    RB = 1024

    def body(h_ref, w_ref, ab_ref, hwx_ref, a_ref):
        hw = h_ref[...] @ w_ref[...]
        a_ref[...] = hw @ ab_ref[...]
        tail = jnp.where(lax.broadcasted_iota(jnp.int32, (RB, 16), 1) == 0,
                         1.0, 0.0).astype(jnp.float32)
        for c in range(2):
            for hh in range(H):
                hwx_ref[c, hh, :, 0:64] = hw[:, hh * D + c * 64:
                                             hh * D + c * 64 + 64]
                hwx_ref[c, hh, :, 64:ROW2] = tail

    return pl.pallas_call(
        body,
        grid=(NP // RB,),
        in_specs=[pl.BlockSpec((RB, D), lambda i: (i, 0)),
                  pl.BlockSpec((D, H * D), lambda i: (0, 0)),
                  pl.BlockSpec((H * D, 16), lambda i: (0, 0))],
        out_specs=[pl.BlockSpec((2, H, RB, ROW2), lambda i: (0, 0, i, 0)),
                   pl.BlockSpec((RB, 16), lambda i: (i, 0))],
        out_shape=[jax.ShapeDtypeStruct((2, H, NP, ROW2), jnp.float32),
                   jax.ShapeDtypeStruct((NP, 16), jnp.float32)],
    )(h, gatW, AB)


def _tk_gat_c(aT):
    def body(at_ref, ct_ref):
        at = at_ref[...]
        ms = jnp.max(at[0:H, :], axis=1, keepdims=True)
        z = ms + at[H:2 * H, :]
        ct_ref[...] = jnp.where(z > 0, z, 0.2 * z)

    return pl.pallas_call(
        body, out_shape=jax.ShapeDtypeStruct((H, NP), jnp.float32),
    )(aT)


def _tk_gat_reduce(gout):
    RB = 1024

    def body(a_ref, b_ref, o_ref):
        j = pl.program_id(1)
        va = a_ref[0, 0]
        vb = b_ref[0, 0]
        ca = va[:, 0:64] / (va[:, 64:65] + 1e-30)
        cb = vb[:, 0:64] / (vb[:, 64:65] + 1e-30)
        contrib = jnp.concatenate([ca, cb], axis=1)

        @pl.when(j == 0)
        def _():
            o_ref[...] = contrib

        @pl.when(j > 0)
        def _():
            o_ref[...] = o_ref[...] + contrib

    return pl.pallas_call(
        body,
        grid=(NP // RB, H),
        in_specs=[pl.BlockSpec((1, 1, RB, ROW2), lambda i, j: (0, j, i, 0)),
                  pl.BlockSpec((1, 1, RB, ROW2), lambda i, j: (1, j, i, 0))],
        out_specs=pl.BlockSpec((RB, D), lambda i, j: (i, 0)),
        out_shape=jax.ShapeDtypeStruct((NP, D), jnp.float32),
    )(gout, gout)


def _tk_att_post(s, xin, lin1W, bn1, lin2, lin3, bn2, mask):
    def body(s_ref, x_ref, w1_ref, g1_ref, be1_ref, w2_ref, b2_ref,
             w3_ref, b3_ref, g2_ref, be2_ref, m_ref, o_ref):
        h = x_ref[...] + s_ref[...] @ w1_ref[...]
        mean = jnp.sum(h, 0, keepdims=True) / 10000.0
        var = jnp.sum(h * h, 0, keepdims=True) / 10000.0 - mean * mean
        h = (h - mean) / jnp.sqrt(var + 1e-5) * g1_ref[...] + be1_ref[...]
        h2 = jnp.maximum(h @ w2_ref[...] + b2_ref[...], 0.0)
        h2 = h2 @ w3_ref[...] + b3_ref[...]
        z = h2 + h
        mk = m_ref[...]
        mean2 = jnp.sum(z * mk, 0, keepdims=True) / 10000.0
        var2 = jnp.sum(z * z * mk, 0, keepdims=True) / 10000.0 - mean2 * mean2
        o_ref[...] = ((z - mean2) / jnp.sqrt(var2 + 1e-5) * g2_ref[...]
                      + be2_ref[...]) * mk

    return pl.pallas_call(
        body, out_shape=jax.ShapeDtypeStruct((NP, D), jnp.float32),
    )(s, xin, lin1W, bn1['g'][None], bn1['b'][None], lin2['W'],
      lin2['b'][None], lin3['W'], lin3['b'][None], bn2['g'][None],
      bn2['b'][None], mask)


def _tk_dinv(degp):
    def body(d_ref, o_ref):
        deg = d_ref[0, :, 0:1] + d_ref[1, :, 0:1]
        o_ref[...] = 1.0 / jnp.sqrt(jnp.maximum(deg, 1.0))

    return pl.pallas_call(
        body, out_shape=jax.ShapeDtypeStruct((NP, 1), jnp.float32),
    )(degp)


def _tk_appnp_comb(aout, x0):
    def body(a_ref, b_ref, x_ref, o_ref):
        o_ref[...] = ((1.0 - ALPHA) * (a_ref[...] + b_ref[...])
                      + ALPHA * x_ref[...])

    return pl.pallas_call(
        body, out_shape=jax.ShapeDtypeStruct((NP, D), jnp.float32),
    )(aout[0], aout[1], x0)


def _tk_pool(hJp, bt_col, bt_row, mask, g1W, g1b, g2W, g2b, n1W, n1b, n2W, n2b):
    def body(hj_ref, bc_ref, br_ref, m_ref, g1w_ref, g1b_ref, g2w_ref,
             g2b_ref, n1w_ref, n1b_ref, n2w_ref, n2b_ref, o_ref):
        hj = hj_ref[...]
        mk = m_ref[...]
        ohf = (bc_ref[...] == lax.broadcasted_iota(jnp.int32, (NP, B), 1)
               ).astype(jnp.float32)
        ohtf = (br_ref[...] == lax.broadcasted_iota(jnp.int32, (B, NP), 0)
                ).astype(jnp.float32)
        for p in range(2):
            g = jnp.maximum(hj @ g1w_ref[p] + g1b_ref[p], 0.0)
            g = g @ g2w_ref[p] + g2b_ref[p]
            m_by_b = jnp.max(jnp.where(ohf > 0, g, -1e30), axis=0,
                             keepdims=True)
            m_at_n = jnp.sum(ohf * m_by_b, axis=1, keepdims=True)
            e = jnp.exp(g - m_at_n) * mk
            d_by_b = jnp.sum(ohf * e, axis=0, keepdims=True)
            d_at_n = jnp.sum(ohf * d_by_b, axis=1, keepdims=True)
            a = e / (d_at_n + 1e-16)
            hn = jnp.maximum(hj @ n1w_ref[p] + n1b_ref[p], 0.0)
            hn = hn @ n2w_ref[p] + n2b_ref[p]
            o_ref[p] = ohtf @ (a * hn)

    return pl.pallas_call(
        body, out_shape=jax.ShapeDtypeStruct((2, B, D), jnp.float32),
    )(hJp, bt_col, bt_row, mask, g1W, g1b, g2W, g2b, n1W, n1b, n2W, n2b)


def _tk_head(hJp, bt_col, bt_row, mask, ctxp, W1a, W1b, b1, bn1, W2, b2,
             bn2, W3, b3):
    def body(hj_ref, bc_ref, br_ref, m_ref, ctx_ref, w1a_ref, w1b_ref,
             b1_ref, g1_ref, be1_ref, w2_ref, b2_ref, g2_ref, be2_ref,
             w3_ref, b3_ref, o_ref):
        hj = hj_ref[...]
        mk = m_ref[...]
        ohf = (bc_ref[...] == lax.broadcasted_iota(jnp.int32, (NP, B), 1)
               ).astype(jnp.float32)
        ohtf = (br_ref[...] == lax.broadcasted_iota(jnp.int32, (B, NP), 0)
                ).astype(jnp.float32)
        ctxw = ctx_ref[...] @ w1a_ref[...]
        z1 = ohf @ ctxw + hj @ w1b_ref[...] + b1_ref[...]
        mean1 = jnp.sum(z1 * mk, 0, keepdims=True) / 10000.0
        var1 = jnp.sum(z1 * z1 * mk, 0, keepdims=True) / 10000.0 - mean1 * mean1
        z1 = jnp.maximum((z1 - mean1) / jnp.sqrt(var1 + 1e-5) * g1_ref[...]
                         + be1_ref[...], 0.0)
        z2 = z1 @ w2_ref[...] + b2_ref[...]
        mean2 = jnp.sum(z2 * mk, 0, keepdims=True) / 10000.0
        var2 = jnp.sum(z2 * z2 * mk, 0, keepdims=True) / 10000.0 - mean2 * mean2
        z2 = jnp.maximum((z2 - mean2) / jnp.sqrt(var2 + 1e-5) * g2_ref[...]
                         + be2_ref[...], 0.0)
        logit = z2 @ w3_ref[...] + b3_ref[...]
        score = 1.0 / (1.0 + jnp.exp(-logit))
        o_ref[...] = ohtf @ (score * mk)

    return pl.pallas_call(
        body, out_shape=jax.ShapeDtypeStruct((B, 1), jnp.float32),
    )(hJp, bt_col, bt_row, mask, ctxp, W1a, W1b, b1[None], bn1['g'][None],
      bn1['b'][None], W2, b2[None], bn2['g'][None], bn2['b'][None], W3,
      b3[None])


def _zero_rows(zbuf, nrows, ncols):
    zero16 = jnp.zeros((16,), jnp.float32)

    def zb(r, _):
        for j in range(ncols // 16):
            zbuf[r, pl.ds(j * 16, 16)] = zero16
        return 0

    lax.fori_loop(0, nrows, zb, 0)


def _sc_gat(hwx, asrcT, adstT, cT, srcp, dstp):
    @functools.partial(
        pl.kernel,
        out_type=jax.ShapeDtypeStruct((2, H, NP, ROW2), jnp.float32),
        mesh=_MESH,
        compiler_params=pltpu.CompilerParams(needs_layout_passes=False, use_tc_tiling_on_sc=False),
        scratch_types=[
            pltpu.VMEM((CHUNK16,), jnp.int32),
            pltpu.VMEM((CHUNK16,), jnp.int32),
            pltpu.VMEM((NTAB,), jnp.float32),
            pltpu.VMEM((NTAB,), jnp.float32),
            pltpu.VMEM((NTAB,), jnp.float32),
            pltpu.VMEM((KBLK_G,), jnp.int32),
            pltpu.VMEM((KBLK_G,), jnp.int32),
            pltpu.VMEM((KBLK_G,), jnp.int32),
            pltpu.VMEM((KBLK_G,), jnp.int32),
            pltpu.VMEM((KBLK_G,), jnp.float32),
            pltpu.VMEM((KBLK_G, ROW2), jnp.float32),
            pltpu.VMEM((KBLK_G, ROW2), jnp.float32),
            pltpu.VMEM_SHARED((NP, ROW2), jnp.float32),
            pltpu.SemaphoreType.DMA,
            pltpu.SemaphoreType.DMA,
            pltpu.SemaphoreType.DMA,
            pltpu.SemaphoreType.DMA,
        ],
    )
    def k(hwx_ref, asrc_ref, adst_ref, c_ref, src_ref, dst_ref, out_ref,
          srcv, dstv, asr, ads, cv, sb0, db0, sb1, db1, wv, gb0, gb1,
          acc, sem0, sem1, sct0, sct1):
        cid = lax.axis_index("c")
        sid = lax.axis_index("s")
        eb = sid * CHUNK16
        pltpu.sync_copy(src_ref.at[pl.ds(eb, CHUNK16)], srcv)
        pltpu.sync_copy(dst_ref.at[pl.ds(eb, CHUNK16)], dstv)

        for hh in range(H):
            pltpu.sync_copy(asrc_ref.at[hh, pl.ds(0, NTAB)], asr)
            pltpu.sync_copy(adst_ref.at[hh, pl.ds(0, NTAB)], ads)
            pltpu.sync_copy(c_ref.at[hh, pl.ds(0, NTAB)], cv)
            _zero_rows(gb0, KBLK_G, ROW2)
            for zi in range(RPS // 40):
                pltpu.sync_copy(gb0.at[pl.ds(0, 40)],
                                acc.at[pl.ds(sid * RPS + zi * 40, 40)])
            plsc.subcore_barrier()

            def prep(sb, db, b):
                off = b * KBLK_G
                for g in range(KBLK_G // 16):
                    sb[pl.ds(g * 16, 16)] = srcv[pl.ds(off + g * 16, 16)]
                    db[pl.ds(g * 16, 16)] = dstv[pl.ds(off + g * 16, 16)]

            def fire(sb, gb, sem):
                pltpu.async_copy(hwx_ref.at[cid].at[hh].at[sb], gb, sem)

            def wait(sb, gb, sem):
                pltpu.make_async_copy(hwx_ref.at[cid].at[hh].at[sb], gb,
                                      sem).wait()

            def compute(sb, db, gb):
                for g in range(KBLK_G // 16):
                    s16 = sb[pl.ds(g * 16, 16)]
                    d16 = db[pl.ds(g * 16, 16)]
                    av = plsc.load_gather(asr, [s16])
                    bv = plsc.load_gather(ads, [d16])
                    cvv = plsc.load_gather(cv, [d16])
                    z = av + bv
                    e = jnp.where(z > 0, z, 0.2 * z)
                    wv[pl.ds(g * 16, 16)] = jnp.exp(e - cvv)

                def sgrp(g2, _):
                    w16 = wv[pl.ds(g2 * 16, 16)]
                    for i in range(16):
                        ws = w16[i]
                        r = g2 * 16 + i
                        for j in range(ROW2 // 16):
                            gb[r, pl.ds(j * 16, 16)] = (
                                gb[r, pl.ds(j * 16, 16)] * ws)
                    return 0

                lax.fori_loop(0, KBLK_G // 16, sgrp, 0)

            def fire_s(db, gb, sct):
                pltpu.async_copy(gb, acc.at[db], sct, add=True)

            def wait_s(db, gb, sct):
                pltpu.make_async_copy(gb, acc.at[db], sct).wait()

            prep(sb0, db0, 0)
            fire(sb0, gb0, sem0)
            _zero_rows(gb1, KBLK_G, ROW2)
            prep(sb1, db1, 0)
            fire_s(db1, gb1, sct1)

            def pair(i, _):
                wait_s(db1, gb1, sct1)
                prep(sb1, db1, 2 * i + 1)
                fire(sb1, gb1, sem1)
                wait(sb0, gb0, sem0)
                compute(sb0, db0, gb0)
                fire_s(db0, gb0, sct0)

                @pl.when(i < NPAIR_G - 1)
                def _():
                    wait_s(db0, gb0, sct0)
                    prep(sb0, db0, 2 * i + 2)
                    fire(sb0, gb0, sem0)

                wait(sb1, gb1, sem1)
                compute(sb1, db1, gb1)
                fire_s(db1, gb1, sct1)
                return 0

            lax.fori_loop(0, NPAIR_G, pair, 0)
            wait_s(db0, gb0, sct0)
            wait_s(db1, gb1, sct1)
            plsc.subcore_barrier()
            pltpu.sync_copy(acc.at[pl.ds(sid * RPS, RPS)],
                            out_ref.at[cid, hh, pl.ds(sid * RPS, RPS)])
            plsc.subcore_barrier()

    return k(hwx, asrcT, adstT, cT, srcp, dstp)


def _sc_deg(dstp):
    @functools.partial(
        pl.kernel,
        out_type=jax.ShapeDtypeStruct((2, NP, 16), jnp.float32),
        mesh=_MESH,
        compiler_params=pltpu.CompilerParams(needs_layout_passes=False, use_tc_tiling_on_sc=False),
        scratch_types=[
            pltpu.VMEM((CHUNK,), jnp.int32),
            pltpu.VMEM((KBLK,), jnp.int32),
            pltpu.VMEM((KBLK, 16), jnp.float32),
            pltpu.VMEM_SHARED((NP, 16), jnp.float32),
        ],
    )
    def k(dst_ref, out_ref, dstv, dbuf, obuf, acc):
        cid = lax.axis_index("c")
        sid = lax.axis_index("s")
        wid = sid * 2 + cid
        pltpu.sync_copy(dst_ref.at[pl.ds(wid * CHUNK, CHUNK)], dstv)
        _zero_rows(obuf, KBLK, 16)
        for zi in range(RPS // KBLK):
            pltpu.sync_copy(obuf, acc.at[pl.ds(sid * RPS + zi * KBLK, KBLK)])
        plsc.subcore_barrier()
        onecol = jnp.where(lax.iota(jnp.int32, 16) == 0, 1.0, 0.0
                           ).astype(jnp.float32)

        def ob(r, _):
            obuf[r, pl.ds(0, 16)] = onecol
            return 0

        lax.fori_loop(0, KBLK, ob, 0)

        def blk(b, _):
            off = b * KBLK
            for g in range(KBLK // 16):
                dbuf[pl.ds(g * 16, 16)] = dstv[pl.ds(off + g * 16, 16)]
            pltpu.sync_copy(obuf, acc.at[dbuf], add=True)
            return 0

        lax.fori_loop(0, NBLK, blk, 0)
        plsc.subcore_barrier()
        pltpu.sync_copy(acc.at[pl.ds(sid * RPS, RPS)],
                        out_ref.at[cid, pl.ds(sid * RPS, RPS)])

    return k(dstp)


def _sc_appnp(hx, dinv_row, srcp, dstp):
    @functools.partial(
        pl.kernel,
        out_type=jax.ShapeDtypeStruct((2, NP, D), jnp.float32),
        mesh=_MESH,
        compiler_params=pltpu.CompilerParams(needs_layout_passes=False, use_tc_tiling_on_sc=False),
        scratch_types=[
            pltpu.VMEM((CHUNK,), jnp.int32),
            pltpu.VMEM((CHUNK,), jnp.int32),
            pltpu.VMEM((NTAB,), jnp.float32),
            pltpu.VMEM((KBLK,), jnp.int32),
            pltpu.VMEM((KBLK,), jnp.int32),
            pltpu.VMEM((KBLK,), jnp.int32),
            pltpu.VMEM((KBLK,), jnp.int32),
            pltpu.VMEM((KBLK,), jnp.float32),
            pltpu.VMEM((KBLK, D), jnp.float32),
            pltpu.VMEM((KBLK, D), jnp.float32),
            pltpu.VMEM_SHARED((NP, D), jnp.float32),
            pltpu.SemaphoreType.DMA,
            pltpu.SemaphoreType.DMA,
            pltpu.SemaphoreType.DMA,
            pltpu.SemaphoreType.DMA,
        ],
    )
    def k(hx_ref, dinv_ref, src_ref, dst_ref, out_ref,
          srcv, dstv, dv, sb0, db0, sb1, db1, wv, gb0, gb1, acc,
          sem0, sem1, sct0, sct1):
        cid = lax.axis_index("c")
        sid = lax.axis_index("s")
        wid = sid * 2 + cid
        eb = wid * CHUNK
        pltpu.sync_copy(src_ref.at[pl.ds(eb, CHUNK)], srcv)
        pltpu.sync_copy(dst_ref.at[pl.ds(eb, CHUNK)], dstv)
        pltpu.sync_copy(dinv_ref.at[0, pl.ds(0, NTAB)], dv)
        _zero_rows(gb0, KBLK, D)
        for zi in range(RPS // KBLK):
            pltpu.sync_copy(gb0, acc.at[pl.ds(sid * RPS + zi * KBLK, KBLK)])
        plsc.subcore_barrier()

        def prep(sb, db, b):
            off = b * KBLK
            for g in range(KBLK // 16):
                sb[pl.ds(g * 16, 16)] = srcv[pl.ds(off + g * 16, 16)]
                db[pl.ds(g * 16, 16)] = dstv[pl.ds(off + g * 16, 16)]

        def fire(sb, gb, sem):
            pltpu.async_copy(hx_ref.at[sb], gb, sem)

        def wait(sb, gb, sem):
            pltpu.make_async_copy(hx_ref.at[sb], gb, sem).wait()

        def compute(sb, db, gb):
            for g in range(KBLK // 16):
                s16 = sb[pl.ds(g * 16, 16)]
                d16 = db[pl.ds(g * 16, 16)]
                wv[pl.ds(g * 16, 16)] = (plsc.load_gather(dv, [s16])
                                         * plsc.load_gather(dv, [d16]))

            def sgrp(g2, _):
                w16 = wv[pl.ds(g2 * 16, 16)]
                for i in range(16):
                    ws = w16[i]
                    r = g2 * 16 + i
                    for j in range(D // 16):
                        gb[r, pl.ds(j * 16, 16)] = (
                            gb[r, pl.ds(j * 16, 16)] * ws)
                return 0

            lax.fori_loop(0, KBLK // 16, sgrp, 0)

        def fire_s(db, gb, sct):
            pltpu.async_copy(gb, acc.at[db], sct, add=True)

        def wait_s(db, gb, sct):
            pltpu.make_async_copy(gb, acc.at[db], sct).wait()

        prep(sb0, db0, 0)
        fire(sb0, gb0, sem0)
        _zero_rows(gb1, KBLK, D)
        prep(sb1, db1, 0)
        fire_s(db1, gb1, sct1)

        def pair(i, _):
            wait_s(db1, gb1, sct1)
            prep(sb1, db1, 2 * i + 1)
            fire(sb1, gb1, sem1)
            wait(sb0, gb0, sem0)
            compute(sb0, db0, gb0)
            fire_s(db0, gb0, sct0)

            @pl.when(i < NPAIR_A - 1)
            def _():
                wait_s(db0, gb0, sct0)
                prep(sb0, db0, 2 * i + 2)
                fire(sb0, gb0, sem0)

            wait(sb1, gb1, sem1)
            compute(sb1, db1, gb1)
            fire_s(db1, gb1, sct1)
            return 0

        lax.fori_loop(0, NPAIR_A, pair, 0)
        wait_s(db0, gb0, sct0)
        wait_s(db1, gb1, sct1)
        plsc.subcore_barrier()
        pltpu.sync_copy(acc.at[pl.ds(sid * RPS, RPS)],
                        out_ref.at[cid, pl.ds(sid * RPS, RPS)])

    return k(hx, dinv_row, srcp, dstp)


def _build_AB(att_src, att_dst):
    idx = jnp.arange(H * D)
    head = (idx // D).astype(jnp.int32)
    A = jnp.zeros((H * D, 16), jnp.float32)
    A = A.at[idx, head].set(att_src.reshape(-1))
    A = A.at[idx, H + head].set(att_dst.reshape(-1))
    return A


def kernel(x, edge_index, batch, n_nodes, Omegas, Phis, Lambdas, Omegas_norm,
           Phis_norm, Lambdas_norm, J, params):
    f32 = jnp.float32
    loop = jnp.arange(N, dtype=edge_index.dtype)
    pad_e = ETP - E_TOT
    srcp = jnp.concatenate([edge_index[0], loop,
                            jnp.full((pad_e,), N, jnp.int32)])
    dstp = jnp.concatenate([edge_index[1], loop,
                            jnp.full((pad_e,), N, jnp.int32)])
    mask = jnp.pad(jnp.ones((N, 1), f32), ((0, NP - N), (0, 0)))
    xJp = jnp.pad(jnp.concatenate([x, J], 1), ((0, NP - N), (0, 7)))
    W1p = jnp.pad(params['lin1_enc']['W'], ((0, 7), (0, 0)))

    h = _tk_encoder(xJp, W1p, params['lin1_enc']['b'], mask)

    for lp in params['att']:
        AB = _build_AB(lp['att_src'], lp['att_dst'])
        hwx, a = _tk_gat_pre(h, lp['gat_W'], AB)
        aT = jnp.transpose(a)
        cT = _tk_gat_c(aT)
        gout = _sc_gat(hwx, aT[0:H], aT[H:2 * H], cT, srcp, dstp)
        s = _tk_gat_reduce(gout)
        h = _tk_att_post(s, h, lp['lin1_W'], lp['bn1'], lp['lin2'],
                         lp['lin3'], lp['bn2'], mask)

    degp = _sc_deg(dstp)
    dinv_row = jnp.transpose(_tk_dinv(degp))
    x0 = h
    for _ in range(KAPPNP):
        aout = _sc_appnp(h, dinv_row, srcp, dstp)
        h = _tk_appnp_comb(aout, x0)

    Jp = jnp.pad(J, ((0, NP - N), (0, 0)))
    hJp = jnp.pad(jnp.concatenate([h, Jp], 1), ((0, 0), (0, 7)))
    batchp = jnp.concatenate([batch,
                              jnp.full((NP - N,), B, batch.dtype)])
    bt_col = batchp[:, None]
    bt_row = batchp[None, :]

    pool = params['pool']
    g1W = jnp.stack([jnp.pad(p['gate1']['W'], ((0, 7), (0, 0)))
                     for p in pool])
    g1b = jnp.stack([p['gate1']['b'][None] for p in pool])
    g2W = jnp.stack([p['gate2']['W'] for p in pool])
    g2b = jnp.stack([p['gate2']['b'][None] for p in pool])
    n1W = jnp.stack([jnp.pad(p['nn1']['W'], ((0, 7), (0, 0))) for p in pool])
    n1b = jnp.stack([p['nn1']['b'][None] for p in pool])
    n2W = jnp.stack([p['nn2']['W'] for p in pool])
    n2b = jnp.stack([p['nn2']['b'][None] for p in pool])
    pools = _tk_pool(hJp, bt_col, bt_row, mask, g1W, g1b, g2W, g2b,
                     n1W, n1b, n2W, n2b)

    ctx = jnp.concatenate([pools[0], pools[1], n_nodes, Omegas, Phis,
                           Lambdas, Omegas_norm, Phis_norm, Lambdas_norm], 1)
    ctxp = jnp.pad(ctx, ((0, 0), (0, 1)))
    W1 = params['head_lin1']['W']
    W1a = jnp.pad(W1[0:263], ((0, 1), (0, 0)))
    W1b = jnp.pad(W1[263:392], ((0, 7), (0, 0)))

    out = _tk_head(hJp, bt_col, bt_row, mask, ctxp, W1a, W1b,
                   params['head_lin1']['b'], params['head_bn1'],
                   params['head_lin2']['W'], params['head_lin2']['b'],
                   params['head_bn2'], params['head_lin3']['W'],
                   params['head_lin3']['b'])
    return out


# speedup vs baseline: 1.0081x; 1.0081x over previous
import functools

import jax
import jax.numpy as jnp
from jax import lax
from jax.experimental import pallas as pl
from jax.experimental.pallas import tpu as pltpu
from jax.experimental.pallas import tpu_sc as plsc

N = 10000
NP = 10240
B = 16
H = 8
D = 128
ROW = 144
E_TOT = 330000
NWORK = 32
CHUNK = 10368
ETP = NWORK * CHUNK
KBLK = 64
NBLK = CHUNK // KBLK
NPAIR_A = NBLK // 2
CHUNK16 = ETP // 16
KBLK_G = 48
NPAIR_G = CHUNK16 // KBLK_G // 2
ROW2 = 80
NTAB = 10048
RPS = NP // 16
ALPHA = 0.1
KAPPNP = 5

_MESH = plsc.VectorSubcoreMesh(core_axis_name="c", subcore_axis_name="s")


def _tk_encoder(xJp, Wp, b, mask):
    def body(x_ref, w_ref, b_ref, m_ref, o_ref):
        o_ref[...] = (x_ref[...] @ w_ref[...] + b_ref[...]) * m_ref[...]
    return pl.pallas_call(
        body, out_shape=jax.ShapeDtypeStruct((NP, D), jnp.float32),
    )(xJp, Wp, b[None], mask)


def _tk_gat_pre(h, gatW, AB):
    RB = 1024

    def body(h_ref, w_ref, ab_ref, hwx_ref, a_ref):
        hw = h_ref[...] @ w_ref[...]
        a_ref[...] = hw @ ab_ref[...]
        tail = jnp.where(lax.broadcasted_iota(jnp.int32, (RB, 16), 1) == 0,
                         1.0, 0.0).astype(jnp.float32)
        for c in range(2):
            for hh in range(H):
                hwx_ref[c, hh, :, 0:64] = hw[:, hh * D + c * 64:
                                             hh * D + c * 64 + 64]
                hwx_ref[c, hh, :, 64:ROW2] = tail

    return pl.pallas_call(
        body,
        grid=(NP // RB,),
        in_specs=[pl.BlockSpec((RB, D), lambda i: (i, 0)),
                  pl.BlockSpec((D, H * D), lambda i: (0, 0)),
                  pl.BlockSpec((H * D, 16), lambda i: (0, 0))],
        out_specs=[pl.BlockSpec((2, H, RB, ROW2), lambda i: (0, 0, i, 0)),
                   pl.BlockSpec((RB, 16), lambda i: (i, 0))],
        out_shape=[jax.ShapeDtypeStruct((2, H, NP, ROW2), jnp.float32),
                   jax.ShapeDtypeStruct((NP, 16), jnp.float32)],
    )(h, gatW, AB)


def _tk_gat_c(aT):
    def body(at_ref, ct_ref):
        at = at_ref[...]
        ms = jnp.max(at[0:H, :], axis=1, keepdims=True)
        z = ms + at[H:2 * H, :]
        ct_ref[...] = jnp.where(z > 0, z, 0.2 * z)

    return pl.pallas_call(
        body, out_shape=jax.ShapeDtypeStruct((H, NP), jnp.float32),
    )(aT)


def _tk_gat_reduce(gout):
    RB = 1024

    def body(a_ref, b_ref, o_ref):
        j = pl.program_id(1)
        va = a_ref[0, 0]
        vb = b_ref[0, 0]
        ca = va[:, 0:64] / (va[:, 64:65] + 1e-30)
        cb = vb[:, 0:64] / (vb[:, 64:65] + 1e-30)
        contrib = jnp.concatenate([ca, cb], axis=1)

        @pl.when(j == 0)
        def _():
            o_ref[...] = contrib

        @pl.when(j > 0)
        def _():
            o_ref[...] = o_ref[...] + contrib

    return pl.pallas_call(
        body,
        grid=(NP // RB, H),
        in_specs=[pl.BlockSpec((1, 1, RB, ROW2), lambda i, j: (0, j, i, 0)),
                  pl.BlockSpec((1, 1, RB, ROW2), lambda i, j: (1, j, i, 0))],
        out_specs=pl.BlockSpec((RB, D), lambda i, j: (i, 0)),
        out_shape=jax.ShapeDtypeStruct((NP, D), jnp.float32),
    )(gout, gout)


def _tk_att_post(s, xin, lin1W, bn1, lin2, lin3, bn2, mask):
    def body(s_ref, x_ref, w1_ref, g1_ref, be1_ref, w2_ref, b2_ref,
             w3_ref, b3_ref, g2_ref, be2_ref, m_ref, o_ref):
        h = x_ref[...] + s_ref[...] @ w1_ref[...]
        mean = jnp.sum(h, 0, keepdims=True) / 10000.0
        var = jnp.sum(h * h, 0, keepdims=True) / 10000.0 - mean * mean
        h = (h - mean) / jnp.sqrt(var + 1e-5) * g1_ref[...] + be1_ref[...]
        h2 = jnp.maximum(h @ w2_ref[...] + b2_ref[...], 0.0)
        h2 = h2 @ w3_ref[...] + b3_ref[...]
        z = h2 + h
        mk = m_ref[...]
        mean2 = jnp.sum(z * mk, 0, keepdims=True) / 10000.0
        var2 = jnp.sum(z * z * mk, 0, keepdims=True) / 10000.0 - mean2 * mean2
        o_ref[...] = ((z - mean2) / jnp.sqrt(var2 + 1e-5) * g2_ref[...]
                      + be2_ref[...]) * mk

    return pl.pallas_call(
        body, out_shape=jax.ShapeDtypeStruct((NP, D), jnp.float32),
    )(s, xin, lin1W, bn1['g'][None], bn1['b'][None], lin2['W'],
      lin2['b'][None], lin3['W'], lin3['b'][None], bn2['g'][None],
      bn2['b'][None], mask)


def _tk_dinv(degp):
    def body(d_ref, o_ref):
        deg = d_ref[0, :, 0:1] + d_ref[1, :, 0:1]
        o_ref[...] = 1.0 / jnp.sqrt(jnp.maximum(deg, 1.0))

    return pl.pallas_call(
        body, out_shape=jax.ShapeDtypeStruct((NP, 1), jnp.float32),
    )(degp)


def _tk_appnp_comb(aout, x0):
    def body(a_ref, b_ref, x_ref, o_ref):
        o_ref[...] = ((1.0 - ALPHA) * (a_ref[...] + b_ref[...])
                      + ALPHA * x_ref[...])

    return pl.pallas_call(
        body, out_shape=jax.ShapeDtypeStruct((NP, D), jnp.float32),
    )(aout[0], aout[1], x0)


def _tk_pool(hJp, bt_col, bt_row, mask, g1W, g1b, g2W, g2b, n1W, n1b, n2W, n2b):
    def body(hj_ref, bc_ref, br_ref, m_ref, g1w_ref, g1b_ref, g2w_ref,
             g2b_ref, n1w_ref, n1b_ref, n2w_ref, n2b_ref, o_ref):
        hj = hj_ref[...]
        mk = m_ref[...]
        ohf = (bc_ref[...] == lax.broadcasted_iota(jnp.int32, (NP, B), 1)
               ).astype(jnp.float32)
        ohtf = (br_ref[...] == lax.broadcasted_iota(jnp.int32, (B, NP), 0)
                ).astype(jnp.float32)
        for p in range(2):
            g = jnp.maximum(hj @ g1w_ref[p] + g1b_ref[p], 0.0)
            g = g @ g2w_ref[p] + g2b_ref[p]
            m_by_b = jnp.max(jnp.where(ohf > 0, g, -1e30), axis=0,
                             keepdims=True)
            m_at_n = jnp.sum(ohf * m_by_b, axis=1, keepdims=True)
            e = jnp.exp(g - m_at_n) * mk
            d_by_b = jnp.sum(ohf * e, axis=0, keepdims=True)
            d_at_n = jnp.sum(ohf * d_by_b, axis=1, keepdims=True)
            a = e / (d_at_n + 1e-16)
            hn = jnp.maximum(hj @ n1w_ref[p] + n1b_ref[p], 0.0)
            hn = hn @ n2w_ref[p] + n2b_ref[p]
            o_ref[p] = ohtf @ (a * hn)

    return pl.pallas_call(
        body, out_shape=jax.ShapeDtypeStruct((2, B, D), jnp.float32),
    )(hJp, bt_col, bt_row, mask, g1W, g1b, g2W, g2b, n1W, n1b, n2W, n2b)


def _tk_head(hJp, bt_col, bt_row, mask, ctxp, W1a, W1b, b1, bn1, W2, b2,
             bn2, W3, b3):
    def body(hj_ref, bc_ref, br_ref, m_ref, ctx_ref, w1a_ref, w1b_ref,
             b1_ref, g1_ref, be1_ref, w2_ref, b2_ref, g2_ref, be2_ref,
             w3_ref, b3_ref, o_ref):
        hj = hj_ref[...]
        mk = m_ref[...]
        ohf = (bc_ref[...] == lax.broadcasted_iota(jnp.int32, (NP, B), 1)
               ).astype(jnp.float32)
        ohtf = (br_ref[...] == lax.broadcasted_iota(jnp.int32, (B, NP), 0)
                ).astype(jnp.float32)
        ctxw = ctx_ref[...] @ w1a_ref[...]
        z1 = ohf @ ctxw + hj @ w1b_ref[...] + b1_ref[...]
        mean1 = jnp.sum(z1 * mk, 0, keepdims=True) / 10000.0
        var1 = jnp.sum(z1 * z1 * mk, 0, keepdims=True) / 10000.0 - mean1 * mean1
        z1 = jnp.maximum((z1 - mean1) / jnp.sqrt(var1 + 1e-5) * g1_ref[...]
                         + be1_ref[...], 0.0)
        z2 = z1 @ w2_ref[...] + b2_ref[...]
        mean2 = jnp.sum(z2 * mk, 0, keepdims=True) / 10000.0
        var2 = jnp.sum(z2 * z2 * mk, 0, keepdims=True) / 10000.0 - mean2 * mean2
        z2 = jnp.maximum((z2 - mean2) / jnp.sqrt(var2 + 1e-5) * g2_ref[...]
                         + be2_ref[...], 0.0)
        logit = z2 @ w3_ref[...] + b3_ref[...]
        score = 1.0 / (1.0 + jnp.exp(-logit))
        o_ref[...] = ohtf @ (score * mk)

    return pl.pallas_call(
        body, out_shape=jax.ShapeDtypeStruct((B, 1), jnp.float32),
    )(hJp, bt_col, bt_row, mask, ctxp, W1a, W1b, b1[None], bn1['g'][None],
      bn1['b'][None], W2, b2[None], bn2['g'][None], bn2['b'][None], W3,
      b3[None])


def _zero_rows(zbuf, nrows, ncols):
    zero16 = jnp.zeros((16,), jnp.float32)

    def zb(r, _):
        for j in range(ncols // 16):
            zbuf[r, pl.ds(j * 16, 16)] = zero16
        return 0

    lax.fori_loop(0, nrows, zb, 0)


def _sc_gat(hwx, asrcT, adstT, cT, srcp, dstp):
    @functools.partial(
        pl.kernel,
        out_type=jax.ShapeDtypeStruct((2, H, NP, ROW2), jnp.float32),
        mesh=_MESH,
        compiler_params=pltpu.CompilerParams(needs_layout_passes=False, use_tc_tiling_on_sc=False),
        scratch_types=[
            pltpu.VMEM((CHUNK16,), jnp.int32),
            pltpu.VMEM((CHUNK16,), jnp.int32),
            pltpu.VMEM((NTAB,), jnp.float32),
            pltpu.VMEM((NTAB,), jnp.float32),
            pltpu.VMEM((NTAB,), jnp.float32),
            pltpu.VMEM((KBLK_G,), jnp.int32),
            pltpu.VMEM((KBLK_G,), jnp.int32),
            pltpu.VMEM((KBLK_G,), jnp.int32),
            pltpu.VMEM((KBLK_G,), jnp.int32),
            pltpu.VMEM((KBLK_G,), jnp.float32),
            pltpu.VMEM((KBLK_G, ROW2), jnp.float32),
            pltpu.VMEM((KBLK_G, ROW2), jnp.float32),
            pltpu.VMEM_SHARED((NP, ROW2), jnp.float32),
            pltpu.SemaphoreType.DMA,
            pltpu.SemaphoreType.DMA,
            pltpu.SemaphoreType.DMA,
            pltpu.SemaphoreType.DMA,
        ],
    )
    def k(hwx_ref, asrc_ref, adst_ref, c_ref, src_ref, dst_ref, out_ref,
          srcv, dstv, asr, ads, cv, sb0, db0, sb1, db1, wv, gb0, gb1,
          acc, sem0, sem1, sct0, sct1):
        cid = lax.axis_index("c")
        sid = lax.axis_index("s")
        eb = sid * CHUNK16
        pltpu.sync_copy(src_ref.at[pl.ds(eb, CHUNK16)], srcv)
        pltpu.sync_copy(dst_ref.at[pl.ds(eb, CHUNK16)], dstv)

        def head_loop(hh, _):
            pltpu.sync_copy(asrc_ref.at[hh, pl.ds(0, NTAB)], asr)
            pltpu.sync_copy(adst_ref.at[hh, pl.ds(0, NTAB)], ads)
            pltpu.sync_copy(c_ref.at[hh, pl.ds(0, NTAB)], cv)
            _zero_rows(gb0, KBLK_G, ROW2)
            for zi in range(RPS // 40):
                pltpu.sync_copy(gb0.at[pl.ds(0, 40)],
                                acc.at[pl.ds(sid * RPS + zi * 40, 40)])
            plsc.subcore_barrier()

            def prep(sb, db, b):
                off = b * KBLK_G
                for g in range(KBLK_G // 16):
                    sb[pl.ds(g * 16, 16)] = srcv[pl.ds(off + g * 16, 16)]
                    db[pl.ds(g * 16, 16)] = dstv[pl.ds(off + g * 16, 16)]

            def fire(sb, gb, sem):
                pltpu.async_copy(hwx_ref.at[cid].at[hh].at[sb], gb, sem)

            def wait(sb, gb, sem):
                pltpu.make_async_copy(hwx_ref.at[cid].at[hh].at[sb], gb,
                                      sem).wait()

            def compute(sb, db, gb):
                for g in range(KBLK_G // 16):
                    s16 = sb[pl.ds(g * 16, 16)]
                    d16 = db[pl.ds(g * 16, 16)]
                    av = plsc.load_gather(asr, [s16])
                    bv = plsc.load_gather(ads, [d16])
                    cvv = plsc.load_gather(cv, [d16])
                    z = av + bv
                    e = jnp.where(z > 0, z, 0.2 * z)
                    wv[pl.ds(g * 16, 16)] = jnp.exp(e - cvv)

                for g2 in range(KBLK_G // 16):
                    w16 = wv[pl.ds(g2 * 16, 16)]
                    for i in range(16):
                        ws = w16[i]
                        r = g2 * 16 + i
                        for j in range(ROW2 // 16):
                            gb[r, pl.ds(j * 16, 16)] = (
                                gb[r, pl.ds(j * 16, 16)] * ws)

            def fire_s(db, gb, sct):
                pltpu.async_copy(gb, acc.at[db], sct, add=True)

            def wait_s(db, gb, sct):
                pltpu.make_async_copy(gb, acc.at[db], sct).wait()

            prep(sb0, db0, 0)
            fire(sb0, gb0, sem0)
            _zero_rows(gb1, KBLK_G, ROW2)
            prep(sb1, db1, 0)
            fire_s(db1, gb1, sct1)

            def pair(i, _):
                wait_s(db1, gb1, sct1)
                prep(sb1, db1, 2 * i + 1)
                fire(sb1, gb1, sem1)
                wait(sb0, gb0, sem0)
                compute(sb0, db0, gb0)
                fire_s(db0, gb0, sct0)

                @pl.when(i < NPAIR_G - 1)
                def _():
                    wait_s(db0, gb0, sct0)
                    prep(sb0, db0, 2 * i + 2)
                    fire(sb0, gb0, sem0)

                wait(sb1, gb1, sem1)
                compute(sb1, db1, gb1)
                fire_s(db1, gb1, sct1)
                return 0

            lax.fori_loop(0, NPAIR_G, pair, 0)
            wait_s(db0, gb0, sct0)
            wait_s(db1, gb1, sct1)
            plsc.subcore_barrier()
            pltpu.sync_copy(acc.at[pl.ds(sid * RPS, RPS)],
                            out_ref.at[cid, hh, pl.ds(sid * RPS, RPS)])
            plsc.subcore_barrier()
            return 0

        lax.fori_loop(0, H, head_loop, 0)

    return k(hwx, asrcT, adstT, cT, srcp, dstp)


def _sc_deg(dstp):
    @functools.partial(
        pl.kernel,
        out_type=jax.ShapeDtypeStruct((2, NP, 16), jnp.float32),
        mesh=_MESH,
        compiler_params=pltpu.CompilerParams(needs_layout_passes=False, use_tc_tiling_on_sc=False),
        scratch_types=[
            pltpu.VMEM((CHUNK,), jnp.int32),
            pltpu.VMEM((KBLK,), jnp.int32),
            pltpu.VMEM((KBLK, 16), jnp.float32),
            pltpu.VMEM_SHARED((NP, 16), jnp.float32),
        ],
    )
    def k(dst_ref, out_ref, dstv, dbuf, obuf, acc):
        cid = lax.axis_index("c")
        sid = lax.axis_index("s")
        wid = sid * 2 + cid
        pltpu.sync_copy(dst_ref.at[pl.ds(wid * CHUNK, CHUNK)], dstv)
        _zero_rows(obuf, KBLK, 16)
        for zi in range(RPS // KBLK):
            pltpu.sync_copy(obuf, acc.at[pl.ds(sid * RPS + zi * KBLK, KBLK)])
        plsc.subcore_barrier()
        onecol = jnp.where(lax.iota(jnp.int32, 16) == 0, 1.0, 0.0
                           ).astype(jnp.float32)

        def ob(r, _):
            obuf[r, pl.ds(0, 16)] = onecol
            return 0

        lax.fori_loop(0, KBLK, ob, 0)

        def blk(b, _):
            off = b * KBLK
            for g in range(KBLK // 16):
                dbuf[pl.ds(g * 16, 16)] = dstv[pl.ds(off + g * 16, 16)]
            pltpu.sync_copy(obuf, acc.at[dbuf], add=True)
            return 0

        lax.fori_loop(0, NBLK, blk, 0)
        plsc.subcore_barrier()
        pltpu.sync_copy(acc.at[pl.ds(sid * RPS, RPS)],
                        out_ref.at[cid, pl.ds(sid * RPS, RPS)])

    return k(dstp)


def _sc_appnp(hx, dinv_row, srcp, dstp):
    @functools.partial(
        pl.kernel,
        out_type=jax.ShapeDtypeStruct((2, NP, D), jnp.float32),
        mesh=_MESH,
        compiler_params=pltpu.CompilerParams(needs_layout_passes=False, use_tc_tiling_on_sc=False),
        scratch_types=[
            pltpu.VMEM((CHUNK,), jnp.int32),
            pltpu.VMEM((CHUNK,), jnp.int32),
            pltpu.VMEM((NTAB,), jnp.float32),
            pltpu.VMEM((KBLK,), jnp.int32),
            pltpu.VMEM((KBLK,), jnp.int32),
            pltpu.VMEM((KBLK,), jnp.int32),
            pltpu.VMEM((KBLK,), jnp.int32),
            pltpu.VMEM((KBLK,), jnp.float32),
            pltpu.VMEM((KBLK, D), jnp.float32),
            pltpu.VMEM((KBLK, D), jnp.float32),
            pltpu.VMEM_SHARED((NP, D), jnp.float32),
            pltpu.SemaphoreType.DMA,
            pltpu.SemaphoreType.DMA,
            pltpu.SemaphoreType.DMA,
            pltpu.SemaphoreType.DMA,
        ],
    )
    def k(hx_ref, dinv_ref, src_ref, dst_ref, out_ref,
          srcv, dstv, dv, sb0, db0, sb1, db1, wv, gb0, gb1, acc,
          sem0, sem1, sct0, sct1):
        cid = lax.axis_index("c")
        sid = lax.axis_index("s")
        wid = sid * 2 + cid
        eb = wid * CHUNK
        pltpu.sync_copy(src_ref.at[pl.ds(eb, CHUNK)], srcv)
        pltpu.sync_copy(dst_ref.at[pl.ds(eb, CHUNK)], dstv)
        pltpu.sync_copy(dinv_ref.at[0, pl.ds(0, NTAB)], dv)
        _zero_rows(gb0, KBLK, D)
        for zi in range(RPS // KBLK):
            pltpu.sync_copy(gb0, acc.at[pl.ds(sid * RPS + zi * KBLK, KBLK)])
        plsc.subcore_barrier()

        def prep(sb, db, b):
            off = b * KBLK
            for g in range(KBLK // 16):
                sb[pl.ds(g * 16, 16)] = srcv[pl.ds(off + g * 16, 16)]
                db[pl.ds(g * 16, 16)] = dstv[pl.ds(off + g * 16, 16)]

        def fire(sb, gb, sem):
            pltpu.async_copy(hx_ref.at[sb], gb, sem)

        def wait(sb, gb, sem):
            pltpu.make_async_copy(hx_ref.at[sb], gb, sem).wait()

        def compute(sb, db, gb):
            for g in range(KBLK // 16):
                s16 = sb[pl.ds(g * 16, 16)]
                d16 = db[pl.ds(g * 16, 16)]
                wv[pl.ds(g * 16, 16)] = (plsc.load_gather(dv, [s16])
                                         * plsc.load_gather(dv, [d16]))

            for g2 in range(KBLK // 16):
                w16 = wv[pl.ds(g2 * 16, 16)]
                for i in range(16):
                    ws = w16[i]
                    r = g2 * 16 + i
                    for j in range(D // 16):
                        gb[r, pl.ds(j * 16, 16)] = (
                            gb[r, pl.ds(j * 16, 16)] * ws)

        def fire_s(db, gb, sct):
            pltpu.async_copy(gb, acc.at[db], sct, add=True)

        def wait_s(db, gb, sct):
            pltpu.make_async_copy(gb, acc.at[db], sct).wait()

        prep(sb0, db0, 0)
        fire(sb0, gb0, sem0)
        _zero_rows(gb1, KBLK, D)
        prep(sb1, db1, 0)
        fire_s(db1, gb1, sct1)

        def pair(i, _):
            wait_s(db1, gb1, sct1)
            prep(sb1, db1, 2 * i + 1)
            fire(sb1, gb1, sem1)
            wait(sb0, gb0, sem0)
            compute(sb0, db0, gb0)
            fire_s(db0, gb0, sct0)

            @pl.when(i < NPAIR_A - 1)
            def _():
                wait_s(db0, gb0, sct0)
                prep(sb0, db0, 2 * i + 2)
                fire(sb0, gb0, sem0)

            wait(sb1, gb1, sem1)
            compute(sb1, db1, gb1)
            fire_s(db1, gb1, sct1)
            return 0

        lax.fori_loop(0, NPAIR_A, pair, 0)
        wait_s(db0, gb0, sct0)
        wait_s(db1, gb1, sct1)
        plsc.subcore_barrier()
        pltpu.sync_copy(acc.at[pl.ds(sid * RPS, RPS)],
                        out_ref.at[cid, pl.ds(sid * RPS, RPS)])

    return k(hx, dinv_row, srcp, dstp)


def _build_AB(att_src, att_dst):
    idx = jnp.arange(H * D)
    head = (idx // D).astype(jnp.int32)
    A = jnp.zeros((H * D, 16), jnp.float32)
    A = A.at[idx, head].set(att_src.reshape(-1))
    A = A.at[idx, H + head].set(att_dst.reshape(-1))
    return A


def kernel(x, edge_index, batch, n_nodes, Omegas, Phis, Lambdas, Omegas_norm,
           Phis_norm, Lambdas_norm, J, params):
    f32 = jnp.float32
    loop = jnp.arange(N, dtype=edge_index.dtype)
    pad_e = ETP - E_TOT
    srcp = jnp.concatenate([edge_index[0], loop,
                            jnp.full((pad_e,), N, jnp.int32)])
    dstp = jnp.concatenate([edge_index[1], loop,
                            jnp.full((pad_e,), N, jnp.int32)])
    mask = jnp.pad(jnp.ones((N, 1), f32), ((0, NP - N), (0, 0)))
    xJp = jnp.pad(jnp.concatenate([x, J], 1), ((0, NP - N), (0, 7)))
    W1p = jnp.pad(params['lin1_enc']['W'], ((0, 7), (0, 0)))

    h = _tk_encoder(xJp, W1p, params['lin1_enc']['b'], mask)

    for lp in params['att']:
        AB = _build_AB(lp['att_src'], lp['att_dst'])
        hwx, a = _tk_gat_pre(h, lp['gat_W'], AB)
        aT = jnp.transpose(a)
        cT = _tk_gat_c(aT)
        gout = _sc_gat(hwx, aT[0:H], aT[H:2 * H], cT, srcp, dstp)
        s = _tk_gat_reduce(gout)
        h = _tk_att_post(s, h, lp['lin1_W'], lp['bn1'], lp['lin2'],
                         lp['lin3'], lp['bn2'], mask)

    degp = _sc_deg(dstp)
    dinv_row = jnp.transpose(_tk_dinv(degp))
    x0 = h
    for _ in range(KAPPNP):
        aout = _sc_appnp(h, dinv_row, srcp, dstp)
        h = _tk_appnp_comb(aout, x0)

    Jp = jnp.pad(J, ((0, NP - N), (0, 0)))
    hJp = jnp.pad(jnp.concatenate([h, Jp], 1), ((0, 0), (0, 7)))
    batchp = jnp.concatenate([batch,
                              jnp.full((NP - N,), B, batch.dtype)])
    bt_col = batchp[:, None]
    bt_row = batchp[None, :]

    pool = params['pool']
    g1W = jnp.stack([jnp.pad(p['gate1']['W'], ((0, 7), (0, 0)))
                     for p in pool])
    g1b = jnp.stack([p['gate1']['b'][None] for p in pool])
    g2W = jnp.stack([p['gate2']['W'] for p in pool])
    g2b = jnp.stack([p['gate2']['b'][None] for p in pool])
    n1W = jnp.stack([jnp.pad(p['nn1']['W'], ((0, 7), (0, 0))) for p in pool])
    n1b = jnp.stack([p['nn1']['b'][None] for p in pool])
    n2W = jnp.stack([p['nn2']['W'] for p in pool])
    n2b = jnp.stack([p['nn2']['b'][None] for p in pool])
    pools = _tk_pool(hJp, bt_col, bt_row, mask, g1W, g1b, g2W, g2b,
                     n1W, n1b, n2W, n2b)

    ctx = jnp.concatenate([pools[0], pools[1], n_nodes, Omegas, Phis,
                           Lambdas, Omegas_norm, Phis_norm, Lambdas_norm], 1)
    ctxp = jnp.pad(ctx, ((0, 0), (0, 1)))
    W1 = params['head_lin1']['W']
    W1a = jnp.pad(W1[0:263], ((0, 1), (0, 0)))
    W1b = jnp.pad(W1[263:392], ((0, 7), (0, 0)))

    out = _tk_head(hJp, bt_col, bt_row, mask, ctxp, W1a, W1b,
                   params['head_lin1']['b'], params['head_bn1'],
                   params['head_lin2']['W'], params['head_lin2']['b'],
                   params['head_bn2'], params['head_lin3']['W'],
                   params['head_lin3']['b'])
    return out


# speedup vs baseline: 1.0935x; 1.0847x over previous
import functools

import jax
import jax.numpy as jnp
from jax import lax
from jax.experimental import pallas as pl
from jax.experimental.pallas import tpu as pltpu
from jax.experimental.pallas import tpu_sc as plsc

N = 10000
NP = 10240
B = 16
H = 8
D = 128
ROW = 144
E_TOT = 330000
NWORK = 32
CHUNK = 10368
ETP = NWORK * CHUNK
KBLK = 64
NBLK = CHUNK // KBLK
NPAIR_A = NBLK // 2
CHUNK16 = ETP // 16
KBLK_G = 48
NPAIR_G = CHUNK16 // KBLK_G // 2
ROW2 = 64
NTAB = 10048
RPS = NP // 16
ALPHA = 0.1
KAPPNP = 5

_MESH = plsc.VectorSubcoreMesh(core_axis_name="c", subcore_axis_name="s")


def _tk_encoder(xJp, Wp, b, mask):
    def body(x_ref, w_ref, b_ref, m_ref, o_ref):
        o_ref[...] = (x_ref[...] @ w_ref[...] + b_ref[...]) * m_ref[...]
    return pl.pallas_call(
        body, out_shape=jax.ShapeDtypeStruct((NP, D), jnp.float32),
    )(xJp, Wp, b[None], mask)


def _tk_gat_pre(h, gatW, AB):
    RB = 1024

    def body(h_ref, w_ref, ab_ref, hwx_ref, a_ref):
        hw = h_ref[...] @ w_ref[...]
        a_ref[...] = hw @ ab_ref[...]
        for c in range(2):
            for hh in range(H):
                hwx_ref[c, hh, :, 0:64] = hw[:, hh * D + c * 64:
                                             hh * D + c * 64 + 64]

    return pl.pallas_call(
        body,
        grid=(NP // RB,),
        in_specs=[pl.BlockSpec((RB, D), lambda i: (i, 0)),
                  pl.BlockSpec((D, H * D), lambda i: (0, 0)),
                  pl.BlockSpec((H * D, 16), lambda i: (0, 0))],
        out_specs=[pl.BlockSpec((2, H, RB, ROW2), lambda i: (0, 0, i, 0)),
                   pl.BlockSpec((RB, 16), lambda i: (i, 0))],
        out_shape=[jax.ShapeDtypeStruct((2, H, NP, ROW2), jnp.float32),
                   jax.ShapeDtypeStruct((NP, 16), jnp.float32)],
    )(h, gatW, AB)


def _tk_gat_c(aT):
    def body(at_ref, ct_ref):
        at = at_ref[...]
        ms = jnp.max(at[0:H, :], axis=1, keepdims=True)
        z = ms + at[H:2 * H, :]
        ct_ref[...] = jnp.where(z > 0, z, 0.2 * z)

    return pl.pallas_call(
        body, out_shape=jax.ShapeDtypeStruct((H, NP), jnp.float32),
    )(aT)


def _tk_gat_reduce(gout):
    RB = 1024

    def body(a_ref, b_ref, da_ref, db_ref, o_ref):
        j = pl.program_id(1)
        ca = a_ref[0, 0] / (da_ref[0, 0, :, 0:1] + 1e-30)
        cb = b_ref[0, 0] / (db_ref[0, 0, :, 0:1] + 1e-30)
        contrib = jnp.concatenate([ca, cb], axis=1)

        @pl.when(j == 0)
        def _():
            o_ref[...] = contrib

        @pl.when(j > 0)
        def _():
            o_ref[...] = o_ref[...] + contrib

    gnum, gden = gout
    return pl.pallas_call(
        body,
        grid=(NP // RB, H),
        in_specs=[pl.BlockSpec((1, 1, RB, ROW2), lambda i, j: (0, j, i, 0)),
                  pl.BlockSpec((1, 1, RB, ROW2), lambda i, j: (1, j, i, 0)),
                  pl.BlockSpec((1, 1, RB, 16), lambda i, j: (0, j, i, 0)),
                  pl.BlockSpec((1, 1, RB, 16), lambda i, j: (1, j, i, 0))],
        out_specs=pl.BlockSpec((RB, D), lambda i, j: (i, 0)),
        out_shape=jax.ShapeDtypeStruct((NP, D), jnp.float32),
    )(gnum, gnum, gden, gden)


def _tk_att_post(s, xin, lin1W, bn1, lin2, lin3, bn2, mask):
    def body(s_ref, x_ref, w1_ref, g1_ref, be1_ref, w2_ref, b2_ref,
             w3_ref, b3_ref, g2_ref, be2_ref, m_ref, o_ref):
        h = x_ref[...] + s_ref[...] @ w1_ref[...]
        mean = jnp.sum(h, 0, keepdims=True) / 10000.0
        var = jnp.sum(h * h, 0, keepdims=True) / 10000.0 - mean * mean
        h = (h - mean) / jnp.sqrt(var + 1e-5) * g1_ref[...] + be1_ref[...]
        h2 = jnp.maximum(h @ w2_ref[...] + b2_ref[...], 0.0)
        h2 = h2 @ w3_ref[...] + b3_ref[...]
        z = h2 + h
        mk = m_ref[...]
        mean2 = jnp.sum(z * mk, 0, keepdims=True) / 10000.0
        var2 = jnp.sum(z * z * mk, 0, keepdims=True) / 10000.0 - mean2 * mean2
        o_ref[...] = ((z - mean2) / jnp.sqrt(var2 + 1e-5) * g2_ref[...]
                      + be2_ref[...]) * mk

    return pl.pallas_call(
        body, out_shape=jax.ShapeDtypeStruct((NP, D), jnp.float32),
    )(s, xin, lin1W, bn1['g'][None], bn1['b'][None], lin2['W'],
      lin2['b'][None], lin3['W'], lin3['b'][None], bn2['g'][None],
      bn2['b'][None], mask)


def _tk_dinv(degp):
    def body(d_ref, o_ref):
        deg = d_ref[0, :, 0:1] + d_ref[1, :, 0:1]
        o_ref[...] = 1.0 / jnp.sqrt(jnp.maximum(deg, 1.0))

    return pl.pallas_call(
        body, out_shape=jax.ShapeDtypeStruct((NP, 1), jnp.float32),
    )(degp)


def _tk_appnp_comb(aout, x0):
    def body(a_ref, b_ref, x_ref, o_ref):
        o_ref[...] = ((1.0 - ALPHA) * (a_ref[...] + b_ref[...])
                      + ALPHA * x_ref[...])

    return pl.pallas_call(
        body, out_shape=jax.ShapeDtypeStruct((NP, D), jnp.float32),
    )(aout[0], aout[1], x0)


def _tk_pool(hJp, bt_col, bt_row, mask, g1W, g1b, g2W, g2b, n1W, n1b, n2W, n2b):
    def body(hj_ref, bc_ref, br_ref, m_ref, g1w_ref, g1b_ref, g2w_ref,
             g2b_ref, n1w_ref, n1b_ref, n2w_ref, n2b_ref, o_ref):
        hj = hj_ref[...]
        mk = m_ref[...]
        ohf = (bc_ref[...] == lax.broadcasted_iota(jnp.int32, (NP, B), 1)
               ).astype(jnp.float32)
        ohtf = (br_ref[...] == lax.broadcasted_iota(jnp.int32, (B, NP), 0)
                ).astype(jnp.float32)
        for p in range(2):
            g = jnp.maximum(hj @ g1w_ref[p] + g1b_ref[p], 0.0)
            g = g @ g2w_ref[p] + g2b_ref[p]
            m_by_b = jnp.max(jnp.where(ohf > 0, g, -1e30), axis=0,
                             keepdims=True)
            m_at_n = jnp.sum(ohf * m_by_b, axis=1, keepdims=True)
            e = jnp.exp(g - m_at_n) * mk
            d_by_b = jnp.sum(ohf * e, axis=0, keepdims=True)
            d_at_n = jnp.sum(ohf * d_by_b, axis=1, keepdims=True)
            a = e / (d_at_n + 1e-16)
            hn = jnp.maximum(hj @ n1w_ref[p] + n1b_ref[p], 0.0)
            hn = hn @ n2w_ref[p] + n2b_ref[p]
            o_ref[p] = ohtf @ (a * hn)

    return pl.pallas_call(
        body, out_shape=jax.ShapeDtypeStruct((2, B, D), jnp.float32),
    )(hJp, bt_col, bt_row, mask, g1W, g1b, g2W, g2b, n1W, n1b, n2W, n2b)


def _tk_head(hJp, bt_col, bt_row, mask, ctxp, W1a, W1b, b1, bn1, W2, b2,
             bn2, W3, b3):
    def body(hj_ref, bc_ref, br_ref, m_ref, ctx_ref, w1a_ref, w1b_ref,
             b1_ref, g1_ref, be1_ref, w2_ref, b2_ref, g2_ref, be2_ref,
             w3_ref, b3_ref, o_ref):
        hj = hj_ref[...]
        mk = m_ref[...]
        ohf = (bc_ref[...] == lax.broadcasted_iota(jnp.int32, (NP, B), 1)
               ).astype(jnp.float32)
        ohtf = (br_ref[...] == lax.broadcasted_iota(jnp.int32, (B, NP), 0)
                ).astype(jnp.float32)
        ctxw = ctx_ref[...] @ w1a_ref[...]
        z1 = ohf @ ctxw + hj @ w1b_ref[...] + b1_ref[...]
        mean1 = jnp.sum(z1 * mk, 0, keepdims=True) / 10000.0
        var1 = jnp.sum(z1 * z1 * mk, 0, keepdims=True) / 10000.0 - mean1 * mean1
        z1 = jnp.maximum((z1 - mean1) / jnp.sqrt(var1 + 1e-5) * g1_ref[...]
                         + be1_ref[...], 0.0)
        z2 = z1 @ w2_ref[...] + b2_ref[...]
        mean2 = jnp.sum(z2 * mk, 0, keepdims=True) / 10000.0
        var2 = jnp.sum(z2 * z2 * mk, 0, keepdims=True) / 10000.0 - mean2 * mean2
        z2 = jnp.maximum((z2 - mean2) / jnp.sqrt(var2 + 1e-5) * g2_ref[...]
                         + be2_ref[...], 0.0)
        logit = z2 @ w3_ref[...] + b3_ref[...]
        score = 1.0 / (1.0 + jnp.exp(-logit))
        o_ref[...] = ohtf @ (score * mk)

    return pl.pallas_call(
        body, out_shape=jax.ShapeDtypeStruct((B, 1), jnp.float32),
    )(hJp, bt_col, bt_row, mask, ctxp, W1a, W1b, b1[None], bn1['g'][None],
      bn1['b'][None], W2, b2[None], bn2['g'][None], bn2['b'][None], W3,
      b3[None])


def _zero_rows(zbuf, nrows, ncols):
    zero16 = jnp.zeros((16,), jnp.float32)

    def zb(r, _):
        for j in range(ncols // 16):
            zbuf[r, pl.ds(j * 16, 16)] = zero16
        return 0

    lax.fori_loop(0, nrows, zb, 0)


def _sc_gat(hwx, asrcT, adstT, cT, srcp, dstp):
    @functools.partial(
        pl.kernel,
        out_type=[jax.ShapeDtypeStruct((2, H, NP, ROW2), jnp.float32),
                  jax.ShapeDtypeStruct((2, H, NP, 16), jnp.float32)],
        mesh=_MESH,
        compiler_params=pltpu.CompilerParams(needs_layout_passes=False, use_tc_tiling_on_sc=False),
        scratch_types=[
            pltpu.VMEM((CHUNK16,), jnp.int32),
            pltpu.VMEM((CHUNK16,), jnp.int32),
            pltpu.VMEM((NTAB,), jnp.float32),
            pltpu.VMEM((NTAB,), jnp.float32),
            pltpu.VMEM((NTAB,), jnp.float32),
            pltpu.VMEM((KBLK_G,), jnp.int32),
            pltpu.VMEM((KBLK_G,), jnp.int32),
            pltpu.VMEM((KBLK_G,), jnp.int32),
            pltpu.VMEM((KBLK_G,), jnp.int32),
            pltpu.VMEM((KBLK_G,), jnp.float32),
            pltpu.VMEM((KBLK_G, ROW2), jnp.float32),
            pltpu.VMEM((KBLK_G, ROW2), jnp.float32),
            pltpu.VMEM((KBLK_G, 16), jnp.float32),
            pltpu.VMEM((KBLK_G, 16), jnp.float32),
            pltpu.VMEM_SHARED((NP, ROW2), jnp.float32),
            pltpu.VMEM_SHARED((NP, 16), jnp.float32),
            pltpu.SemaphoreType.DMA,
            pltpu.SemaphoreType.DMA,
            pltpu.SemaphoreType.DMA,
            pltpu.SemaphoreType.DMA,
            pltpu.SemaphoreType.DMA,
            pltpu.SemaphoreType.DMA,
        ],
    )
    def k(hwx_ref, asrc_ref, adst_ref, c_ref, src_ref, dst_ref,
          outn_ref, outd_ref,
          srcv, dstv, asr, ads, cv, sb0, db0, sb1, db1, wv, gb0, gb1,
          wc0, wc1, acc, dacc, sem0, sem1, sct0, sct1, sdn0, sdn1):
        cid = lax.axis_index("c")
        sid = lax.axis_index("s")
        eb = sid * CHUNK16
        pltpu.sync_copy(src_ref.at[pl.ds(eb, CHUNK16)], srcv)
        pltpu.sync_copy(dst_ref.at[pl.ds(eb, CHUNK16)], dstv)
        onecol = jnp.where(lax.iota(jnp.int32, 16) == 0, 1.0, 0.0
                           ).astype(jnp.float32)

        def head_loop(hh, _):
            pltpu.sync_copy(asrc_ref.at[hh, pl.ds(0, NTAB)], asr)
            pltpu.sync_copy(adst_ref.at[hh, pl.ds(0, NTAB)], ads)
            pltpu.sync_copy(c_ref.at[hh, pl.ds(0, NTAB)], cv)
            _zero_rows(gb0, KBLK_G, ROW2)
            _zero_rows(wc0, KBLK_G, 16)
            for zi in range(RPS // 40):
                pltpu.sync_copy(gb0.at[pl.ds(0, 40)],
                                acc.at[pl.ds(sid * RPS + zi * 40, 40)])
                pltpu.sync_copy(wc0.at[pl.ds(0, 40)],
                                dacc.at[pl.ds(sid * RPS + zi * 40, 40)])
            plsc.subcore_barrier()

            def prep(sb, db, b):
                off = b * KBLK_G
                for g in range(KBLK_G // 16):
                    sb[pl.ds(g * 16, 16)] = srcv[pl.ds(off + g * 16, 16)]
                    db[pl.ds(g * 16, 16)] = dstv[pl.ds(off + g * 16, 16)]

            def fire(sb, gb, sem):
                pltpu.async_copy(hwx_ref.at[cid].at[hh].at[sb], gb, sem)

            def wait(sb, gb, sem):
                pltpu.make_async_copy(hwx_ref.at[cid].at[hh].at[sb], gb,
                                      sem).wait()

            def compute(sb, db, gb, wc):
                for g in range(KBLK_G // 16):
                    s16 = sb[pl.ds(g * 16, 16)]
                    d16 = db[pl.ds(g * 16, 16)]
                    av = plsc.load_gather(asr, [s16])
                    bv = plsc.load_gather(ads, [d16])
                    cvv = plsc.load_gather(cv, [d16])
                    z = av + bv
                    e = jnp.where(z > 0, z, 0.2 * z)
                    wv[pl.ds(g * 16, 16)] = jnp.exp(e - cvv)

                for g2 in range(KBLK_G // 16):
                    w16 = wv[pl.ds(g2 * 16, 16)]
                    for i in range(16):
                        ws = w16[i]
                        r = g2 * 16 + i
                        for j in range(ROW2 // 16):
                            gb[r, pl.ds(j * 16, 16)] = (
                                gb[r, pl.ds(j * 16, 16)] * ws)
                        wc[r, pl.ds(0, 16)] = onecol * ws

            def fire_s(db, gb, wc, sct, sdn):
                pltpu.async_copy(gb, acc.at[db], sct, add=True)
                pltpu.async_copy(wc, dacc.at[db], sdn, add=True)

            def wait_s(db, gb, wc, sct, sdn):
                pltpu.make_async_copy(gb, acc.at[db], sct).wait()
                pltpu.make_async_copy(wc, dacc.at[db], sdn).wait()

            prep(sb0, db0, 0)
            fire(sb0, gb0, sem0)
            _zero_rows(gb1, KBLK_G, ROW2)
            _zero_rows(wc1, KBLK_G, 16)
            prep(sb1, db1, 0)
            fire_s(db1, gb1, wc1, sct1, sdn1)

            def pair(i, _):
                wait_s(db1, gb1, wc1, sct1, sdn1)
                prep(sb1, db1, 2 * i + 1)
                fire(sb1, gb1, sem1)
                wait(sb0, gb0, sem0)
                compute(sb0, db0, gb0, wc0)
                fire_s(db0, gb0, wc0, sct0, sdn0)

                @pl.when(i < NPAIR_G - 1)
                def _():
                    wait_s(db0, gb0, wc0, sct0, sdn0)
                    prep(sb0, db0, 2 * i + 2)
                    fire(sb0, gb0, sem0)

                wait(sb1, gb1, sem1)
                compute(sb1, db1, gb1, wc1)
                fire_s(db1, gb1, wc1, sct1, sdn1)
                return 0

            lax.fori_loop(0, NPAIR_G, pair, 0)
            wait_s(db0, gb0, wc0, sct0, sdn0)
            wait_s(db1, gb1, wc1, sct1, sdn1)
            plsc.subcore_barrier()
            pltpu.sync_copy(acc.at[pl.ds(sid * RPS, RPS)],
                            outn_ref.at[cid, hh, pl.ds(sid * RPS, RPS)])
            pltpu.sync_copy(dacc.at[pl.ds(sid * RPS, RPS)],
                            outd_ref.at[cid, hh, pl.ds(sid * RPS, RPS)])
            plsc.subcore_barrier()
            return 0

        lax.fori_loop(0, H, head_loop, 0)

    return k(hwx, asrcT, adstT, cT, srcp, dstp)


def _sc_deg(dstp):
    @functools.partial(
        pl.kernel,
        out_type=jax.ShapeDtypeStruct((2, NP, 16), jnp.float32),
        mesh=_MESH,
        compiler_params=pltpu.CompilerParams(needs_layout_passes=False, use_tc_tiling_on_sc=False),
        scratch_types=[
            pltpu.VMEM((CHUNK,), jnp.int32),
            pltpu.VMEM((KBLK,), jnp.int32),
            pltpu.VMEM((KBLK, 16), jnp.float32),
            pltpu.VMEM_SHARED((NP, 16), jnp.float32),
        ],
    )
    def k(dst_ref, out_ref, dstv, dbuf, obuf, acc):
        cid = lax.axis_index("c")
        sid = lax.axis_index("s")
        wid = sid * 2 + cid
        pltpu.sync_copy(dst_ref.at[pl.ds(wid * CHUNK, CHUNK)], dstv)
        _zero_rows(obuf, KBLK, 16)
        for zi in range(RPS // KBLK):
            pltpu.sync_copy(obuf, acc.at[pl.ds(sid * RPS + zi * KBLK, KBLK)])
        plsc.subcore_barrier()
        onecol = jnp.where(lax.iota(jnp.int32, 16) == 0, 1.0, 0.0
                           ).astype(jnp.float32)

        def ob(r, _):
            obuf[r, pl.ds(0, 16)] = onecol
            return 0

        lax.fori_loop(0, KBLK, ob, 0)

        def blk(b, _):
            off = b * KBLK
            for g in range(KBLK // 16):
                dbuf[pl.ds(g * 16, 16)] = dstv[pl.ds(off + g * 16, 16)]
            pltpu.sync_copy(obuf, acc.at[dbuf], add=True)
            return 0

        lax.fori_loop(0, NBLK, blk, 0)
        plsc.subcore_barrier()
        pltpu.sync_copy(acc.at[pl.ds(sid * RPS, RPS)],
                        out_ref.at[cid, pl.ds(sid * RPS, RPS)])

    return k(dstp)


def _sc_appnp(hx, dinv_row, srcp, dstp):
    @functools.partial(
        pl.kernel,
        out_type=jax.ShapeDtypeStruct((2, NP, D), jnp.float32),
        mesh=_MESH,
        compiler_params=pltpu.CompilerParams(needs_layout_passes=False, use_tc_tiling_on_sc=False),
        scratch_types=[
            pltpu.VMEM((CHUNK,), jnp.int32),
            pltpu.VMEM((CHUNK,), jnp.int32),
            pltpu.VMEM((NTAB,), jnp.float32),
            pltpu.VMEM((KBLK,), jnp.int32),
            pltpu.VMEM((KBLK,), jnp.int32),
            pltpu.VMEM((KBLK,), jnp.int32),
            pltpu.VMEM((KBLK,), jnp.int32),
            pltpu.VMEM((KBLK,), jnp.float32),
            pltpu.VMEM((KBLK, D), jnp.float32),
            pltpu.VMEM((KBLK, D), jnp.float32),
            pltpu.VMEM_SHARED((NP, D), jnp.float32),
            pltpu.SemaphoreType.DMA,
            pltpu.SemaphoreType.DMA,
            pltpu.SemaphoreType.DMA,
            pltpu.SemaphoreType.DMA,
        ],
    )
    def k(hx_ref, dinv_ref, src_ref, dst_ref, out_ref,
          srcv, dstv, dv, sb0, db0, sb1, db1, wv, gb0, gb1, acc,
          sem0, sem1, sct0, sct1):
        cid = lax.axis_index("c")
        sid = lax.axis_index("s")
        wid = sid * 2 + cid
        eb = wid * CHUNK
        pltpu.sync_copy(src_ref.at[pl.ds(eb, CHUNK)], srcv)
        pltpu.sync_copy(dst_ref.at[pl.ds(eb, CHUNK)], dstv)
        pltpu.sync_copy(dinv_ref.at[0, pl.ds(0, NTAB)], dv)
        _zero_rows(gb0, KBLK, D)
        for zi in range(RPS // KBLK):
            pltpu.sync_copy(gb0, acc.at[pl.ds(sid * RPS + zi * KBLK, KBLK)])
        plsc.subcore_barrier()

        def prep(sb, db, b):
            off = b * KBLK
            for g in range(KBLK // 16):
                sb[pl.ds(g * 16, 16)] = srcv[pl.ds(off + g * 16, 16)]
                db[pl.ds(g * 16, 16)] = dstv[pl.ds(off + g * 16, 16)]

        def fire(sb, gb, sem):
            pltpu.async_copy(hx_ref.at[sb], gb, sem)

        def wait(sb, gb, sem):
            pltpu.make_async_copy(hx_ref.at[sb], gb, sem).wait()

        def compute(sb, db, gb):
            for g in range(KBLK // 16):
                s16 = sb[pl.ds(g * 16, 16)]
                d16 = db[pl.ds(g * 16, 16)]
                wv[pl.ds(g * 16, 16)] = (plsc.load_gather(dv, [s16])
                                         * plsc.load_gather(dv, [d16]))

            for g2 in range(KBLK // 16):
                w16 = wv[pl.ds(g2 * 16, 16)]
                for i in range(16):
                    ws = w16[i]
                    r = g2 * 16 + i
                    for j in range(D // 16):
                        gb[r, pl.ds(j * 16, 16)] = (
                            gb[r, pl.ds(j * 16, 16)] * ws)

        def fire_s(db, gb, sct):
            pltpu.async_copy(gb, acc.at[db], sct, add=True)

        def wait_s(db, gb, sct):
            pltpu.make_async_copy(gb, acc.at[db], sct).wait()

        prep(sb0, db0, 0)
        fire(sb0, gb0, sem0)
        _zero_rows(gb1, KBLK, D)
        prep(sb1, db1, 0)
        fire_s(db1, gb1, sct1)

        def pair(i, _):
            wait_s(db1, gb1, sct1)
            prep(sb1, db1, 2 * i + 1)
            fire(sb1, gb1, sem1)
            wait(sb0, gb0, sem0)
            compute(sb0, db0, gb0)
            fire_s(db0, gb0, sct0)

            @pl.when(i < NPAIR_A - 1)
            def _():
                wait_s(db0, gb0, sct0)
                prep(sb0, db0, 2 * i + 2)
                fire(sb0, gb0, sem0)

            wait(sb1, gb1, sem1)
            compute(sb1, db1, gb1)
            fire_s(db1, gb1, sct1)
            return 0

        lax.fori_loop(0, NPAIR_A, pair, 0)
        wait_s(db0, gb0, sct0)
        wait_s(db1, gb1, sct1)
        plsc.subcore_barrier()
        pltpu.sync_copy(acc.at[pl.ds(sid * RPS, RPS)],
                        out_ref.at[cid, pl.ds(sid * RPS, RPS)])

    return k(hx, dinv_row, srcp, dstp)


def _build_AB(att_src, att_dst):
    idx = jnp.arange(H * D)
    head = (idx // D).astype(jnp.int32)
    A = jnp.zeros((H * D, 16), jnp.float32)
    A = A.at[idx, head].set(att_src.reshape(-1))
    A = A.at[idx, H + head].set(att_dst.reshape(-1))
    return A


def kernel(x, edge_index, batch, n_nodes, Omegas, Phis, Lambdas, Omegas_norm,
           Phis_norm, Lambdas_norm, J, params):
    f32 = jnp.float32
    loop = jnp.arange(N, dtype=edge_index.dtype)
    pad_e = ETP - E_TOT
    srcp = jnp.concatenate([edge_index[0], loop,
                            jnp.full((pad_e,), N, jnp.int32)])
    dstp = jnp.concatenate([edge_index[1], loop,
                            jnp.full((pad_e,), N, jnp.int32)])
    mask = jnp.pad(jnp.ones((N, 1), f32), ((0, NP - N), (0, 0)))
    xJp = jnp.pad(jnp.concatenate([x, J], 1), ((0, NP - N), (0, 7)))
    W1p = jnp.pad(params['lin1_enc']['W'], ((0, 7), (0, 0)))

    h = _tk_encoder(xJp, W1p, params['lin1_enc']['b'], mask)

    for lp in params['att']:
        AB = _build_AB(lp['att_src'], lp['att_dst'])
        hwx, a = _tk_gat_pre(h, lp['gat_W'], AB)
        aT = jnp.transpose(a)
        cT = _tk_gat_c(aT)
        gout = _sc_gat(hwx, aT[0:H], aT[H:2 * H], cT, srcp, dstp)
        s = _tk_gat_reduce(gout)
        h = _tk_att_post(s, h, lp['lin1_W'], lp['bn1'], lp['lin2'],
                         lp['lin3'], lp['bn2'], mask)

    degp = _sc_deg(dstp)
    dinv_row = jnp.transpose(_tk_dinv(degp))
    x0 = h
    for _ in range(KAPPNP):
        aout = _sc_appnp(h, dinv_row, srcp, dstp)
        h = _tk_appnp_comb(aout, x0)

    Jp = jnp.pad(J, ((0, NP - N), (0, 0)))
    hJp = jnp.pad(jnp.concatenate([h, Jp], 1), ((0, 0), (0, 7)))
    batchp = jnp.concatenate([batch,
                              jnp.full((NP - N,), B, batch.dtype)])
    bt_col = batchp[:, None]
    bt_row = batchp[None, :]

    pool = params['pool']
    g1W = jnp.stack([jnp.pad(p['gate1']['W'], ((0, 7), (0, 0)))
                     for p in pool])
    g1b = jnp.stack([p['gate1']['b'][None] for p in pool])
    g2W = jnp.stack([p['gate2']['W'] for p in pool])
    g2b = jnp.stack([p['gate2']['b'][None] for p in pool])
    n1W = jnp.stack([jnp.pad(p['nn1']['W'], ((0, 7), (0, 0))) for p in pool])
    n1b = jnp.stack([p['nn1']['b'][None] for p in pool])
    n2W = jnp.stack([p['nn2']['W'] for p in pool])
    n2b = jnp.stack([p['nn2']['b'][None] for p in pool])
    pools = _tk_pool(hJp, bt_col, bt_row, mask, g1W, g1b, g2W, g2b,
                     n1W, n1b, n2W, n2b)

    ctx = jnp.concatenate([pools[0], pools[1], n_nodes, Omegas, Phis,
                           Lambdas, Omegas_norm, Phis_norm, Lambdas_norm], 1)
    ctxp = jnp.pad(ctx, ((0, 0), (0, 1)))
    W1 = params['head_lin1']['W']
    W1a = jnp.pad(W1[0:263], ((0, 1), (0, 0)))
    W1b = jnp.pad(W1[263:392], ((0, 7), (0, 0)))

    out = _tk_head(hJp, bt_col, bt_row, mask, ctxp, W1a, W1b,
                   params['head_lin1']['b'], params['head_bn1'],
                   params['head_lin2']['W'], params['head_lin2']['b'],
                   params['head_bn2'], params['head_lin3']['W'],
                   params['head_lin3']['b'])
    return out
